# baseline (speedup 1.0000x reference)
import functools

import jax
import jax.numpy as jnp
from jax import lax
from jax.experimental import pallas as pl
from jax.experimental.pallas import tpu as pltpu
from jax.experimental.pallas import tpu_sc as plsc

_EPS = 1e-5


def _softplus(x):
    return jnp.maximum(x, 0.0) + jnp.log1p(jnp.exp(-jnp.abs(x)))


def _sigmoid(x):
    return 0.5 * jnp.tanh(0.5 * x) + 0.5


def _sc_gather(table, idx_flat, chunk=400):
    n_rows, d = table.shape
    b = idx_flat.shape[0]
    info = plsc.get_sparse_core_info()
    nw = info.num_cores * info.num_subcores
    per_w = b // nw
    while per_w % chunk != 0:
        chunk //= 2
    assert per_w * nw == b and per_w % chunk == 0 and chunk % 8 == 0
    n_chunks = per_w // chunk
    n_pairs = n_chunks // 2
    tail = n_chunks % 2
    mesh = plsc.VectorSubcoreMesh(core_axis_name="c", subcore_axis_name="s")

    @functools.partial(
        pl.kernel,
        mesh=mesh,
        out_type=jax.ShapeDtypeStruct((b, d), table.dtype),
        compiler_params=pltpu.CompilerParams(use_tc_tiling_on_sc=True),
        scratch_types=[
            pltpu.VMEM((chunk,), jnp.int32),
            pltpu.VMEM((chunk,), jnp.int32),
            pltpu.VMEM((chunk, d), table.dtype),
            pltpu.VMEM((chunk, d), table.dtype),
            pltpu.SemaphoreType.DMA,
            pltpu.SemaphoreType.DMA,
            pltpu.SemaphoreType.DMA,
        ],
    )
    def k(table_hbm, idx_hbm, out_hbm, idx_v0, idx_v1, rows_v0, rows_v1,
          sem_g, sem_o0, sem_o1):
        wid = lax.axis_index("s") * info.num_cores + lax.axis_index("c")
        base = pl.multiple_of(wid * per_w, 8)
        idx_v = (idx_v0, idx_v1)
        rows_v = (rows_v0, rows_v1)
        sem_o = (sem_o0, sem_o1)

        def chunk_step(ci, sl, guard):
            off = pl.multiple_of(base + ci * chunk, 8)

            @pl.when(guard)
            def _drain():
                pltpu.make_async_copy(
                    rows_v[sl], out_hbm.at[pl.ds(off, chunk)],
                    sem_o[sl]).wait()

            pltpu.sync_copy(idx_hbm.at[pl.ds(off, chunk)], idx_v[sl])
            pltpu.async_copy(table_hbm.at[idx_v[sl]], rows_v[sl],
                             sem_g).wait()
            pltpu.async_copy(rows_v[sl],
                             out_hbm.at[pl.ds(off, chunk)], sem_o[sl])

        def body(j, _):
            for sl in range(2):
                chunk_step(2 * j + sl, sl, j > 0)
            return ()

        lax.fori_loop(0, n_pairs, body, (), unroll=False)
        if tail:
            chunk_step(2 * n_pairs, 0, n_pairs > 0)
        for sl in range(2):
            if (n_pairs > 0) or (tail and sl == 0):
                pltpu.make_async_copy(rows_v[sl],
                                      out_hbm.at[pl.ds(base, chunk)],
                                      sem_o[sl]).wait()

    return k(table, idx_flat)


def _center_body(atom_ref, wc_ref, bf_ref, out_ref):
    out_ref[...] = (
        jnp.dot(atom_ref[...], wc_ref[...], preferred_element_type=jnp.float32)
        + bf_ref[...]
    )


def _edge_preact(g, q, wg, wn):
    return (
        jnp.dot(g.astype(jnp.bfloat16), wg, preferred_element_type=jnp.float32)
        + jnp.dot(q, wn, preferred_element_type=jnp.float32)
    )


def _stats_body(hb, ga_ref, gb_ref, q_ref, a_ref, wg_ref, wn_ref, stats_ref):
    step = pl.program_id(0)

    @pl.when(step == 0)
    def _init():
        stats_ref[...] = jnp.zeros_like(stats_ref)

    a = a_ref[...]
    q = q_ref[...].astype(jnp.bfloat16)
    g = jnp.where(step < hb, ga_ref[...], gb_ref[...])
    x2 = _edge_preact(g, q, wg_ref[...], wn_ref[...])
    ba, hf = a.shape
    m = x2.shape[0] // ba
    t = jnp.sum(x2.reshape(ba, m, hf), axis=1)
    s = jnp.sum(t, axis=0) + m * jnp.sum(a, axis=0)
    sq = (
        jnp.sum(x2 * x2, axis=0)
        + 2.0 * jnp.sum(a * t, axis=0)
        + m * jnp.sum(a * a, axis=0)
    )
    stats_ref[0:1, :] += s.reshape(1, hf)
    stats_ref[1:2, :] += sq.reshape(1, hf)


def _apply_body(count, hb, ga_ref, gb_ref, q_ref, a_ref, w_ref, wg_ref,
                wn_ref, stats_ref, g1_ref, b1_ref, s_out_ref, st2_ref,
                wgs_scr, wns_scr):
    step = pl.program_id(0)

    @pl.when(step == 0)
    def _init_stats():
        st2_ref[...] = jnp.zeros_like(st2_ref)

    mu = stats_ref[0:1, :] / count
    ex2 = stats_ref[1:2, :] / count
    var = ex2 - mu * mu
    inv = lax.rsqrt(var + _EPS)
    scale = g1_ref[...] * inv
    shift = b1_ref[...] - mu * scale

    @pl.when(step == 0)
    def _scale_weights():
        wgs_scr[...] = (wg_ref[...] * scale.astype(jnp.bfloat16))
        wns_scr[...] = (wn_ref[...] * scale.astype(jnp.bfloat16))

    q = q_ref[...].astype(jnp.bfloat16)
    g = jnp.where(step < hb, ga_ref[...], gb_ref[...])
    y2 = _edge_preact(g, q, wgs_scr[...], wns_scr[...])
    a = a_ref[...]
    ba, hf = a.shape
    m = y2.shape[0] // ba
    af = hf // 2
    a2 = a * scale + shift
    y3 = y2.reshape(ba, m, hf) + a2[:, None, :]
    f = _sigmoid(y3[:, :, :af])
    c = _softplus(y3[:, :, af:])
    w = w_ref[...]
    prod = f * c * (w * w)[:, :, None]
    s_blk = jnp.sum(prod, axis=1)
    s_out_ref[...] = s_blk
    st2_ref[0:1, :] += jnp.sum(s_blk, axis=0).reshape(1, af)
    st2_ref[1:2, :] += jnp.sum(s_blk * s_blk, axis=0).reshape(1, af)


def _final_body(count, s_ref, atom_ref, st2_ref, g2_ref,
                b2_ref, wp_ref, bp_ref, out_ref):
    mu = st2_ref[0:1, :] / count
    ex2 = st2_ref[1:2, :] / count
    var = ex2 - mu * mu
    inv = lax.rsqrt(var + _EPS)
    scale = g2_ref[...] * inv
    shift = b2_ref[...] - mu * scale
    h = _softplus(atom_ref[...] + s_ref[...] * scale + shift)
    out_ref[...] = (
        jnp.dot(h, wp_ref[...], preferred_element_type=jnp.float32)
        + bp_ref[...]
    )


def _tc_pipeline(atom_fea, g0a, g0b, nbr_flat, bond_w, W_full, b_full,
                 g1, b1, g2, b2, W_proj, b_proj):
    n, af = atom_fea.shape
    nm = 2 * g0a.shape[0]
    m = nm // n
    nf = nbr_flat.shape[1]
    hf = 2 * af

    wc = W_full[:af]
    wg = W_full[af:2 * af].astype(jnp.bfloat16)
    wn = W_full[2 * af:].astype(jnp.bfloat16)

    a_center = pl.pallas_call(
        _center_body,
        out_shape=jax.ShapeDtypeStruct((n, hf), jnp.float32),
    )(atom_fea, wc, b_full.reshape(1, hf))

    ba = 200
    be = ba * m
    nsteps = n // ba
    hb = nsteps // 2
    full = lambda shp: pl.BlockSpec(shp, lambda i: (0,) * len(shp))
    ga_spec = pl.BlockSpec((be, af), lambda i: (jnp.minimum(i, hb - 1), 0))
    gb_spec = pl.BlockSpec((be, af), lambda i: (jnp.maximum(i - hb, 0), 0))

    stats = pl.pallas_call(
        functools.partial(_stats_body, hb),
        grid=(nsteps,),
        in_specs=[
            ga_spec,
            gb_spec,
            pl.BlockSpec((be, nf), lambda i: (i, 0)),
            pl.BlockSpec((ba, hf), lambda i: (i, 0)),
            full((af, hf)),
            full((nf, hf)),
        ],
        out_specs=pl.BlockSpec((8, hf), lambda i: (0, 0)),
        out_shape=jax.ShapeDtypeStruct((8, hf), jnp.float32),
    )(g0a, g0b, nbr_flat, a_center, wg, wn)

    s_sum, st2 = pl.pallas_call(
        functools.partial(_apply_body, float(nm), hb),
        grid=(nsteps,),
        in_specs=[
            ga_spec,
            gb_spec,
            pl.BlockSpec((be, nf), lambda i: (i, 0)),
            pl.BlockSpec((ba, hf), lambda i: (i, 0)),
            pl.BlockSpec((ba, m), lambda i: (i, 0)),
            full((af, hf)),
            full((nf, hf)),
            full((8, hf)),
            full((1, hf)),
            full((1, hf)),
        ],
        out_specs=[
            pl.BlockSpec((ba, af), lambda i: (i, 0)),
            pl.BlockSpec((8, af), lambda i: (0, 0)),
        ],
        out_shape=[
            jax.ShapeDtypeStruct((n, af), jnp.float32),
            jax.ShapeDtypeStruct((8, af), jnp.float32),
        ],
        scratch_shapes=[
            pltpu.VMEM((af, hf), jnp.bfloat16),
            pltpu.VMEM((nf, hf), jnp.bfloat16),
        ],
    )(g0a, g0b, nbr_flat, a_center, bond_w, wg, wn, stats,
      g1.reshape(1, hf), b1.reshape(1, hf))

    ba2 = 2000
    atom_out = pl.pallas_call(
        functools.partial(_final_body, float(n)),
        grid=(n // ba2,),
        in_specs=[
            pl.BlockSpec((ba2, af), lambda i: (i, 0)),
            pl.BlockSpec((ba2, af), lambda i: (i, 0)),
            pl.BlockSpec((8, af), lambda i: (0, 0)),
            pl.BlockSpec((1, af), lambda i: (0, 0)),
            pl.BlockSpec((1, af), lambda i: (0, 0)),
            pl.BlockSpec((af, af), lambda i: (0, 0)),
            pl.BlockSpec((1, af), lambda i: (0, 0)),
        ],
        out_specs=pl.BlockSpec((ba2, af), lambda i: (i, 0)),
        out_shape=jax.ShapeDtypeStruct((n, af), jnp.float32),
    )(s_sum, atom_fea, st2, g2.reshape(1, af), b2.reshape(1, af),
      W_proj, b_proj.reshape(1, af))

    return atom_out


def kernel(atom_fea, nbr_fea, nbr_fea_idx, bond_weights_ag,
           W_full, b_full, g1, b1, g2, b2, W_proj, b_proj):
    n, m = nbr_fea_idx.shape
    nf = nbr_fea.shape[2]
    idx_flat = nbr_fea_idx.reshape(n * m).astype(jnp.int32)
    g0a = _sc_gather(atom_fea, idx_flat[: n * m // 2])
    g0b = _sc_gather(atom_fea, idx_flat[n * m // 2:])
    nbr_flat = nbr_fea.reshape(n * m, nf)
    atom_out = _tc_pipeline(atom_fea, g0a, g0b, nbr_flat, bond_weights_ag,
                            W_full, b_full, g1, b1, g2, b2, W_proj, b_proj)
    return atom_out, nbr_fea


# speedup vs baseline: 1.0697x; 1.0697x over previous
import functools

import jax
import jax.numpy as jnp
from jax import lax
from jax.experimental import pallas as pl
from jax.experimental.pallas import tpu as pltpu
from jax.experimental.pallas import tpu_sc as plsc

_EPS = 1e-5


def _softplus(x):
    return jnp.maximum(x, 0.0) + jnp.log1p(jnp.exp(-jnp.abs(x)))


def _sigmoid(x):
    return 0.5 * jnp.tanh(0.5 * x) + 0.5


def _sc_gather(table, idx_flat, chunk=400):
    n_rows, d = table.shape
    b = idx_flat.shape[0]
    info = plsc.get_sparse_core_info()
    nw = info.num_cores * info.num_subcores
    per_w = b // nw
    while per_w % chunk != 0:
        chunk //= 2
    assert per_w * nw == b and per_w % chunk == 0 and chunk % 8 == 0
    n_chunks = per_w // chunk
    assert n_chunks >= 3
    mesh = plsc.VectorSubcoreMesh(core_axis_name="c", subcore_axis_name="s")

    @functools.partial(
        pl.kernel,
        mesh=mesh,
        out_type=jax.ShapeDtypeStruct((b, d), table.dtype),
        compiler_params=pltpu.CompilerParams(use_tc_tiling_on_sc=True),
        scratch_types=[
            pltpu.VMEM((chunk,), jnp.int32),
            pltpu.VMEM((chunk,), jnp.int32),
            pltpu.VMEM((chunk, d), table.dtype),
            pltpu.VMEM((chunk, d), table.dtype),
            pltpu.SemaphoreType.DMA,
            pltpu.SemaphoreType.DMA,
            pltpu.SemaphoreType.DMA,
            pltpu.SemaphoreType.DMA,
        ],
    )
    def k(table_hbm, idx_hbm, out_hbm, idx_v0, idx_v1, rows_v0, rows_v1,
          sem_g0, sem_g1, sem_o0, sem_o1):
        wid = lax.axis_index("s") * info.num_cores + lax.axis_index("c")
        base = pl.multiple_of(wid * per_w, 8)
        idx_v = (idx_v0, idx_v1)
        rows_v = (rows_v0, rows_v1)
        sem_g = (sem_g0, sem_g1)
        sem_o = (sem_o0, sem_o1)

        def off(ci):
            return pl.multiple_of(base + ci * chunk, 8)

        def launch(ci, sl):
            pltpu.sync_copy(idx_hbm.at[pl.ds(off(ci), chunk)], idx_v[sl])
            pltpu.async_copy(table_hbm.at[idx_v[sl]], rows_v[sl], sem_g[sl])

        launch(0, 0)
        launch(1, 1)

        def step(ci, sl, refill):
            pltpu.make_async_copy(table_hbm.at[idx_v[sl]], rows_v[sl],
                                  sem_g[sl]).wait()
            pltpu.async_copy(rows_v[sl], out_hbm.at[pl.ds(off(ci), chunk)],
                             sem_o[sl])

            if refill:
                @pl.when(ci + 2 < n_chunks)
                def _refill():
                    pltpu.make_async_copy(
                        rows_v[sl], out_hbm.at[pl.ds(off(ci), chunk)],
                        sem_o[sl]).wait()
                    launch(ci + 2, sl)

        def body(j, _):
            for sl in range(2):
                step(2 * j + sl, sl, True)
            return ()

        lax.fori_loop(0, n_chunks // 2, body, (), unroll=False)
        if n_chunks % 2:
            step(n_chunks - 1, 0, False)
        for sl in range(2):
            pltpu.make_async_copy(rows_v[sl],
                                  out_hbm.at[pl.ds(base, chunk)],
                                  sem_o[sl]).wait()

    return k(table, idx_flat)


def _center_body(atom_ref, wc_ref, bf_ref, out_ref):
    out_ref[...] = (
        jnp.dot(atom_ref[...], wc_ref[...], preferred_element_type=jnp.float32)
        + bf_ref[...]
    )


def _edge_preact(g, q, wg, wn):
    return (
        jnp.dot(g.astype(jnp.bfloat16), wg, preferred_element_type=jnp.float32)
        + jnp.dot(q, wn, preferred_element_type=jnp.float32)
    )


def _stats_body(g_ref, q_ref, a_ref, wg_ref, wn_ref, stats_ref):
    step = pl.program_id(0)

    @pl.when(step == 0)
    def _init():
        stats_ref[...] = jnp.zeros_like(stats_ref)

    a = a_ref[...]
    q = q_ref[...].astype(jnp.bfloat16)
    x2 = _edge_preact(g_ref[...], q, wg_ref[...], wn_ref[...])
    ba, hf = a.shape
    m = x2.shape[0] // ba
    t = jnp.sum(x2.reshape(ba, m, hf), axis=1)
    s = jnp.sum(t, axis=0) + m * jnp.sum(a, axis=0)
    sq = (
        jnp.sum(x2 * x2, axis=0)
        + 2.0 * jnp.sum(a * t, axis=0)
        + m * jnp.sum(a * a, axis=0)
    )
    stats_ref[0:1, :] += s.reshape(1, hf)
    stats_ref[1:2, :] += sq.reshape(1, hf)


def _apply_body(count, g_ref, q_ref, a_ref, w_ref, wg_ref,
                wn_ref, stats_ref, g1_ref, b1_ref, s_out_ref, st2_ref,
                wgs_scr, wns_scr):
    step = pl.program_id(0)

    @pl.when(step == 0)
    def _init_stats():
        st2_ref[...] = jnp.zeros_like(st2_ref)

    mu = stats_ref[0:1, :] / count
    ex2 = stats_ref[1:2, :] / count
    var = ex2 - mu * mu
    inv = lax.rsqrt(var + _EPS)
    scale = g1_ref[...] * inv
    shift = b1_ref[...] - mu * scale

    @pl.when(step == 0)
    def _scale_weights():
        wgs_scr[...] = (wg_ref[...] * scale.astype(jnp.bfloat16))
        wns_scr[...] = (wn_ref[...] * scale.astype(jnp.bfloat16))

    q = q_ref[...].astype(jnp.bfloat16)
    y2 = _edge_preact(g_ref[...], q, wgs_scr[...], wns_scr[...])
    a = a_ref[...]
    ba, hf = a.shape
    m = y2.shape[0] // ba
    af = hf // 2
    a2 = a * scale + shift
    y3 = y2.reshape(ba, m, hf) + a2[:, None, :]
    f = _sigmoid(y3[:, :, :af])
    c = _softplus(y3[:, :, af:])
    w = w_ref[...]
    prod = f * c * (w * w)[:, :, None]
    s_blk = jnp.sum(prod, axis=1)
    s_out_ref[...] = s_blk
    st2_ref[0:1, :] += jnp.sum(s_blk, axis=0).reshape(1, af)
    st2_ref[1:2, :] += jnp.sum(s_blk * s_blk, axis=0).reshape(1, af)


def _final_body(count, s_ref, atom_ref, st2_ref, g2_ref,
                b2_ref, wp_ref, bp_ref, out_ref):
    mu = st2_ref[0:1, :] / count
    ex2 = st2_ref[1:2, :] / count
    var = ex2 - mu * mu
    inv = lax.rsqrt(var + _EPS)
    scale = g2_ref[...] * inv
    shift = b2_ref[...] - mu * scale
    h = _softplus(atom_ref[...] + s_ref[...] * scale + shift)
    out_ref[...] = (
        jnp.dot(h, wp_ref[...], preferred_element_type=jnp.float32)
        + bp_ref[...]
    )


def _tc_pipeline(atom_fea, g0, nbr_flat, bond_w, W_full, b_full,
                 g1, b1, g2, b2, W_proj, b_proj):
    n, af = atom_fea.shape
    nm = g0.shape[0]
    m = nm // n
    nf = nbr_flat.shape[1]
    hf = 2 * af

    wc = W_full[:af]
    wg = W_full[af:2 * af].astype(jnp.bfloat16)
    wn = W_full[2 * af:].astype(jnp.bfloat16)

    a_center = pl.pallas_call(
        _center_body,
        out_shape=jax.ShapeDtypeStruct((n, hf), jnp.float32),
    )(atom_fea, wc, b_full.reshape(1, hf))

    ba = 200
    be = ba * m
    nsteps = n // ba
    full = lambda shp: pl.BlockSpec(shp, lambda i: (0,) * len(shp))

    stats = pl.pallas_call(
        _stats_body,
        grid=(nsteps,),
        in_specs=[
            pl.BlockSpec((be, af), lambda i: (i, 0)),
            pl.BlockSpec((be, nf), lambda i: (i, 0)),
            pl.BlockSpec((ba, hf), lambda i: (i, 0)),
            full((af, hf)),
            full((nf, hf)),
        ],
        out_specs=pl.BlockSpec((8, hf), lambda i: (0, 0)),
        out_shape=jax.ShapeDtypeStruct((8, hf), jnp.float32),
    )(g0, nbr_flat, a_center, wg, wn)

    s_sum, st2 = pl.pallas_call(
        functools.partial(_apply_body, float(nm)),
        grid=(nsteps,),
        in_specs=[
            pl.BlockSpec((be, af), lambda i: (i, 0)),
            pl.BlockSpec((be, nf), lambda i: (i, 0)),
            pl.BlockSpec((ba, hf), lambda i: (i, 0)),
            pl.BlockSpec((ba, m), lambda i: (i, 0)),
            full((af, hf)),
            full((nf, hf)),
            full((8, hf)),
            full((1, hf)),
            full((1, hf)),
        ],
        out_specs=[
            pl.BlockSpec((ba, af), lambda i: (i, 0)),
            pl.BlockSpec((8, af), lambda i: (0, 0)),
        ],
        out_shape=[
            jax.ShapeDtypeStruct((n, af), jnp.float32),
            jax.ShapeDtypeStruct((8, af), jnp.float32),
        ],
        scratch_shapes=[
            pltpu.VMEM((af, hf), jnp.bfloat16),
            pltpu.VMEM((nf, hf), jnp.bfloat16),
        ],
    )(g0, nbr_flat, a_center, bond_w, wg, wn, stats,
      g1.reshape(1, hf), b1.reshape(1, hf))

    ba2 = 2000
    atom_out = pl.pallas_call(
        functools.partial(_final_body, float(n)),
        grid=(n // ba2,),
        in_specs=[
            pl.BlockSpec((ba2, af), lambda i: (i, 0)),
            pl.BlockSpec((ba2, af), lambda i: (i, 0)),
            full((8, af)),
            full((1, af)),
            full((1, af)),
            full((af, af)),
            full((1, af)),
        ],
        out_specs=pl.BlockSpec((ba2, af), lambda i: (i, 0)),
        out_shape=jax.ShapeDtypeStruct((n, af), jnp.float32),
    )(s_sum, atom_fea, st2, g2.reshape(1, af), b2.reshape(1, af),
      W_proj, b_proj.reshape(1, af))

    return atom_out


def kernel(atom_fea, nbr_fea, nbr_fea_idx, bond_weights_ag,
           W_full, b_full, g1, b1, g2, b2, W_proj, b_proj):
    n, m = nbr_fea_idx.shape
    nf = nbr_fea.shape[2]
    idx_flat = nbr_fea_idx.reshape(n * m).astype(jnp.int32)
    g0 = _sc_gather(atom_fea, idx_flat)
    nbr_flat = nbr_fea.reshape(n * m, nf)
    atom_out = _tc_pipeline(atom_fea, g0, nbr_flat, bond_weights_ag,
                            W_full, b_full, g1, b1, g2, b2, W_proj, b_proj)
    return atom_out, nbr_fea


# speedup vs baseline: 1.0737x; 1.0037x over previous
import functools

import jax
import jax.numpy as jnp
from jax import lax
from jax.experimental import pallas as pl
from jax.experimental.pallas import tpu as pltpu
from jax.experimental.pallas import tpu_sc as plsc

_EPS = 1e-5


def _softplus(x):
    return jnp.maximum(x, 0.0) + jnp.log1p(jnp.exp(-jnp.abs(x)))


def _sigmoid(x):
    return 0.5 * jnp.tanh(0.5 * x) + 0.5


def _sc_gather(table, idx_flat, chunk=400):
    n_rows, d = table.shape
    b = idx_flat.shape[0]
    info = plsc.get_sparse_core_info()
    nw = info.num_cores * info.num_subcores
    per_w = b // nw
    while per_w % chunk != 0:
        chunk //= 2
    assert per_w * nw == b and per_w % chunk == 0 and chunk % 8 == 0
    n_chunks = per_w // chunk
    assert n_chunks >= 3
    mesh = plsc.VectorSubcoreMesh(core_axis_name="c", subcore_axis_name="s")

    @functools.partial(
        pl.kernel,
        mesh=mesh,
        out_type=jax.ShapeDtypeStruct((b, d), table.dtype),
        compiler_params=pltpu.CompilerParams(use_tc_tiling_on_sc=True),
        scratch_types=[
            pltpu.VMEM((chunk,), jnp.int32),
            pltpu.VMEM((chunk,), jnp.int32),
            pltpu.VMEM((chunk, d), table.dtype),
            pltpu.VMEM((chunk, d), table.dtype),
            pltpu.SemaphoreType.DMA,
            pltpu.SemaphoreType.DMA,
            pltpu.SemaphoreType.DMA,
            pltpu.SemaphoreType.DMA,
        ],
    )
    def k(table_hbm, idx_hbm, out_hbm, idx_v0, idx_v1, rows_v0, rows_v1,
          sem_g0, sem_g1, sem_o0, sem_o1):
        wid = lax.axis_index("s") * info.num_cores + lax.axis_index("c")
        base = pl.multiple_of(wid * per_w, 8)
        idx_v = (idx_v0, idx_v1)
        rows_v = (rows_v0, rows_v1)
        sem_g = (sem_g0, sem_g1)
        sem_o = (sem_o0, sem_o1)

        def off(ci):
            return pl.multiple_of(base + ci * chunk, 8)

        def launch(ci, sl):
            pltpu.sync_copy(idx_hbm.at[pl.ds(off(ci), chunk)], idx_v[sl])
            pltpu.async_copy(table_hbm.at[idx_v[sl]], rows_v[sl], sem_g[sl])

        launch(0, 0)
        launch(1, 1)

        def step(ci, sl, refill):
            pltpu.make_async_copy(table_hbm.at[idx_v[sl]], rows_v[sl],
                                  sem_g[sl]).wait()
            pltpu.async_copy(rows_v[sl], out_hbm.at[pl.ds(off(ci), chunk)],
                             sem_o[sl])

            if refill:
                @pl.when(ci + 2 < n_chunks)
                def _refill():
                    pltpu.make_async_copy(
                        rows_v[sl], out_hbm.at[pl.ds(off(ci), chunk)],
                        sem_o[sl]).wait()
                    launch(ci + 2, sl)

        def body(j, _):
            for sl in range(2):
                step(2 * j + sl, sl, True)
            return ()

        lax.fori_loop(0, n_chunks // 2, body, (), unroll=False)
        if n_chunks % 2:
            step(n_chunks - 1, 0, False)
        for sl in range(2):
            pltpu.make_async_copy(rows_v[sl],
                                  out_hbm.at[pl.ds(base, chunk)],
                                  sem_o[sl]).wait()

    return k(table, idx_flat)


def _center_body(atom_ref, wc_ref, bf_ref, out_ref):
    out_ref[...] = (
        jnp.dot(atom_ref[...], wc_ref[...], preferred_element_type=jnp.float32)
        + bf_ref[...]
    )


def _edge_preact(g, q, wg, wn):
    return (
        jnp.dot(g.astype(jnp.bfloat16), wg, preferred_element_type=jnp.float32)
        + jnp.dot(q, wn, preferred_element_type=jnp.float32)
    )


def _fused_body(nsteps, ba, ba2, g_ref, q_ref, a_ref, w_ref, atom_ref,
                wg_ref, wn_ref, g1_ref, b1_ref, g2_ref, b2_ref, wp_ref,
                bp_ref, out_ref,
                stats_scr, st2_scr, wgs_scr, wns_scr, s_scr):
    i = pl.program_id(0)

    @pl.when(i == 0)
    def _init():
        stats_scr[...] = jnp.zeros_like(stats_scr)
        st2_scr[...] = jnp.zeros_like(st2_scr)

    @pl.when(i < nsteps)
    def _stats_phase():
        a = a_ref[...]
        q = q_ref[...].astype(jnp.bfloat16)
        x2 = _edge_preact(g_ref[...], q, wg_ref[...], wn_ref[...])
        ba_, hf = a.shape
        m = x2.shape[0] // ba_
        t = jnp.sum(x2.reshape(ba_, m, hf), axis=1)
        s = jnp.sum(t, axis=0) + m * jnp.sum(a, axis=0)
        sq = (
            jnp.sum(x2 * x2, axis=0)
            + 2.0 * jnp.sum(a * t, axis=0)
            + m * jnp.sum(a * a, axis=0)
        )
        stats_scr[0:1, :] += s.reshape(1, hf)
        stats_scr[1:2, :] += sq.reshape(1, hf)

    @pl.when((i >= nsteps) & (i < 2 * nsteps))
    def _apply_phase():
        a = a_ref[...]
        ba_, hf = a.shape
        af = hf // 2
        m = g_ref.shape[0] // ba_
        count = float(nsteps * ba * m)
        mu = stats_scr[0:1, :] / count
        ex2 = stats_scr[1:2, :] / count
        var = ex2 - mu * mu
        inv = lax.rsqrt(var + _EPS)
        scale = g1_ref[...] * inv
        shift = b1_ref[...] - mu * scale

        @pl.when(i == nsteps)
        def _scale_weights():
            wgs_scr[...] = (wg_ref[...] * scale.astype(jnp.bfloat16))
            wns_scr[...] = (wn_ref[...] * scale.astype(jnp.bfloat16))

        q = q_ref[...].astype(jnp.bfloat16)
        y2 = _edge_preact(g_ref[...], q, wgs_scr[...], wns_scr[...])
        a2 = a * scale + shift
        y3 = y2.reshape(ba_, m, hf) + a2[:, None, :]
        f = _sigmoid(y3[:, :, :af])
        c = _softplus(y3[:, :, af:])
        w = w_ref[...]
        prod = f * c * (w * w)[:, :, None]
        s_blk = jnp.sum(prod, axis=1)
        s_scr[pl.ds((i - nsteps) * ba, ba), :] = s_blk
        st2_scr[0:1, :] += jnp.sum(s_blk, axis=0).reshape(1, af)
        st2_scr[1:2, :] += jnp.sum(s_blk * s_blk, axis=0).reshape(1, af)

    @pl.when(i >= 2 * nsteps)
    def _project_phase():
        j = i - 2 * nsteps
        n_count = float(s_scr.shape[0])
        mu = st2_scr[0:1, :] / n_count
        ex2 = st2_scr[1:2, :] / n_count
        var = ex2 - mu * mu
        inv = lax.rsqrt(var + _EPS)
        scale = g2_ref[...] * inv
        shift = b2_ref[...] - mu * scale
        s = s_scr[pl.ds(j * ba2, ba2), :]
        h = _softplus(atom_ref[...] + s * scale + shift)
        out_ref[...] = (
            jnp.dot(h, wp_ref[...], preferred_element_type=jnp.float32)
            + bp_ref[...]
        )


def _tc_pipeline(atom_fea, g0, nbr_flat, bond_w, W_full, b_full,
                 g1, b1, g2, b2, W_proj, b_proj):
    n, af = atom_fea.shape
    nm = g0.shape[0]
    m = nm // n
    nf = nbr_flat.shape[1]
    hf = 2 * af

    wc = W_full[:af]
    wg = W_full[af:2 * af].astype(jnp.bfloat16)
    wn = W_full[2 * af:].astype(jnp.bfloat16)

    a_center = pl.pallas_call(
        _center_body,
        out_shape=jax.ShapeDtypeStruct((n, hf), jnp.float32),
    )(atom_fea, wc, b_full.reshape(1, hf))

    ba = 200
    be = ba * m
    nsteps = n // ba
    ba2 = 2000
    k4steps = n // ba2
    total = 2 * nsteps + k4steps
    full = lambda shp: pl.BlockSpec(shp, lambda i: (0,) * len(shp))

    def edge_idx(i):
        return jnp.where(i >= 2 * nsteps, nsteps - 1, lax.rem(i, nsteps))

    atom_out = pl.pallas_call(
        functools.partial(_fused_body, nsteps, ba, ba2),
        grid=(total,),
        in_specs=[
            pl.BlockSpec((be, af), lambda i: (edge_idx(i), 0)),
            pl.BlockSpec((be, nf), lambda i: (edge_idx(i), 0)),
            pl.BlockSpec((ba, hf), lambda i: (edge_idx(i), 0)),
            pl.BlockSpec((ba, m), lambda i: (edge_idx(i), 0)),
            pl.BlockSpec((ba2, af),
                         lambda i: (jnp.maximum(i - 2 * nsteps, 0), 0)),
            full((af, hf)),
            full((nf, hf)),
            full((1, hf)),
            full((1, hf)),
            full((1, af)),
            full((1, af)),
            full((af, af)),
            full((1, af)),
        ],
        out_specs=pl.BlockSpec((ba2, af),
                               lambda i: (jnp.maximum(i - 2 * nsteps, 0), 0)),
        out_shape=jax.ShapeDtypeStruct((n, af), jnp.float32),
        scratch_shapes=[
            pltpu.VMEM((8, hf), jnp.float32),
            pltpu.VMEM((8, af), jnp.float32),
            pltpu.VMEM((af, hf), jnp.bfloat16),
            pltpu.VMEM((nf, hf), jnp.bfloat16),
            pltpu.VMEM((n, af), jnp.float32),
        ],
    )(g0, nbr_flat, a_center, bond_w, atom_fea, wg, wn,
      g1.reshape(1, hf), b1.reshape(1, hf), g2.reshape(1, af),
      b2.reshape(1, af), W_proj, b_proj.reshape(1, af))

    return atom_out


def kernel(atom_fea, nbr_fea, nbr_fea_idx, bond_weights_ag,
           W_full, b_full, g1, b1, g2, b2, W_proj, b_proj):
    n, m = nbr_fea_idx.shape
    nf = nbr_fea.shape[2]
    idx_flat = nbr_fea_idx.reshape(n * m).astype(jnp.int32)
    g0 = _sc_gather(atom_fea, idx_flat)
    nbr_flat = nbr_fea.reshape(n * m, nf)
    atom_out = _tc_pipeline(atom_fea, g0, nbr_flat, bond_weights_ag,
                            W_full, b_full, g1, b1, g2, b2, W_proj, b_proj)
    return atom_out, nbr_fea


# speedup vs baseline: 1.1154x; 1.0388x over previous
import functools

import jax
import jax.numpy as jnp
from jax import lax
from jax.experimental import pallas as pl
from jax.experimental.pallas import tpu as pltpu
from jax.experimental.pallas import tpu_sc as plsc

_EPS = 1e-5


def _softplus(x):
    return jnp.maximum(x, 0.0) + jnp.log1p(jnp.exp(-jnp.abs(x)))


def _sigmoid(x):
    return 0.5 * jnp.tanh(0.5 * x) + 0.5


def _sc_gather(table, idx_flat, chunk=400):
    n_rows, d = table.shape
    b = idx_flat.shape[0]
    info = plsc.get_sparse_core_info()
    nw = info.num_cores * info.num_subcores
    per_w = b // nw
    while per_w % chunk != 0:
        chunk //= 2
    assert per_w * nw == b and per_w % chunk == 0 and chunk % 8 == 0
    n_chunks = per_w // chunk
    assert n_chunks >= 3
    mesh = plsc.VectorSubcoreMesh(core_axis_name="c", subcore_axis_name="s")

    @functools.partial(
        pl.kernel,
        mesh=mesh,
        out_type=jax.ShapeDtypeStruct((b, d), table.dtype),
        compiler_params=pltpu.CompilerParams(use_tc_tiling_on_sc=True),
        scratch_types=[
            pltpu.VMEM((chunk,), jnp.int32),
            pltpu.VMEM((chunk,), jnp.int32),
            pltpu.VMEM((chunk, d), table.dtype),
            pltpu.VMEM((chunk, d), table.dtype),
            pltpu.SemaphoreType.DMA,
            pltpu.SemaphoreType.DMA,
            pltpu.SemaphoreType.DMA,
            pltpu.SemaphoreType.DMA,
        ],
    )
    def k(table_hbm, idx_hbm, out_hbm, idx_v0, idx_v1, rows_v0, rows_v1,
          sem_g0, sem_g1, sem_o0, sem_o1):
        wid = lax.axis_index("s") * info.num_cores + lax.axis_index("c")
        base = pl.multiple_of(wid * per_w, 8)
        idx_v = (idx_v0, idx_v1)
        rows_v = (rows_v0, rows_v1)
        sem_g = (sem_g0, sem_g1)
        sem_o = (sem_o0, sem_o1)

        def off(ci):
            return pl.multiple_of(base + ci * chunk, 8)

        def launch(ci, sl):
            pltpu.sync_copy(idx_hbm.at[pl.ds(off(ci), chunk)], idx_v[sl])
            pltpu.async_copy(table_hbm.at[idx_v[sl]], rows_v[sl], sem_g[sl])

        launch(0, 0)
        launch(1, 1)

        def step(ci, sl, refill):
            pltpu.make_async_copy(table_hbm.at[idx_v[sl]], rows_v[sl],
                                  sem_g[sl]).wait()
            pltpu.async_copy(rows_v[sl], out_hbm.at[pl.ds(off(ci), chunk)],
                             sem_o[sl])

            if refill:
                @pl.when(ci + 2 < n_chunks)
                def _refill():
                    pltpu.make_async_copy(
                        rows_v[sl], out_hbm.at[pl.ds(off(ci), chunk)],
                        sem_o[sl]).wait()
                    launch(ci + 2, sl)

        def body(j, _):
            for sl in range(2):
                step(2 * j + sl, sl, True)
            return ()

        lax.fori_loop(0, n_chunks // 2, body, (), unroll=False)
        if n_chunks % 2:
            step(n_chunks - 1, 0, False)
        for sl in range(2):
            pltpu.make_async_copy(rows_v[sl],
                                  out_hbm.at[pl.ds(base, chunk)],
                                  sem_o[sl]).wait()

    return k(table, idx_flat)


def _center_body(atom_ref, wc_ref, bf_ref, out_ref):
    out_ref[...] = (
        jnp.dot(atom_ref[...], wc_ref[...], preferred_element_type=jnp.float32)
        + bf_ref[...]
    )


def _edge_preact(g, q, wg, wn):
    return (
        jnp.dot(g.astype(jnp.bfloat16), wg, preferred_element_type=jnp.float32)
        + jnp.dot(q, wn, preferred_element_type=jnp.float32)
    )


def _fused_body(nsteps, ba, ba2, g_ref, q_ref, a_ref, w_ref, atom_ref,
                wg_ref, wn_ref, g1_ref, b1_ref, g2_ref, b2_ref, wp_ref,
                bp_ref, out_ref,
                stats_scr, st2_scr, wgs_scr, wns_scr, s_scr):
    i = pl.program_id(0)

    @pl.when(i == 0)
    def _init():
        stats_scr[...] = jnp.zeros_like(stats_scr)
        st2_scr[...] = jnp.zeros_like(st2_scr)

    @pl.when(i < nsteps)
    def _stats_phase():
        a = a_ref[...]
        q = q_ref[...].astype(jnp.bfloat16)
        x2 = _edge_preact(g_ref[...], q, wg_ref[...], wn_ref[...])
        ba_, hf = a.shape
        m = x2.shape[0] // ba_
        t = jnp.sum(x2.reshape(ba_, m, hf), axis=1)
        s = jnp.sum(t, axis=0) + m * jnp.sum(a, axis=0)
        sq = (
            jnp.sum(x2 * x2, axis=0)
            + 2.0 * jnp.sum(a * t, axis=0)
            + m * jnp.sum(a * a, axis=0)
        )
        stats_scr[0:1, :] += s.reshape(1, hf)
        stats_scr[1:2, :] += sq.reshape(1, hf)

    @pl.when((i >= nsteps) & (i < 2 * nsteps))
    def _apply_phase():
        a = a_ref[...]
        ba_, hf = a.shape
        af = hf // 2
        m = g_ref.shape[0] // ba_
        count = float(nsteps * ba * m)
        mu = stats_scr[0:1, :] / count
        ex2 = stats_scr[1:2, :] / count
        var = ex2 - mu * mu
        inv = lax.rsqrt(var + _EPS)
        scale = g1_ref[...] * inv
        shift = b1_ref[...] - mu * scale

        @pl.when(i == nsteps)
        def _scale_weights():
            wgs_scr[...] = (wg_ref[...] * scale.astype(jnp.bfloat16))
            wns_scr[...] = (wn_ref[...] * scale.astype(jnp.bfloat16))

        q = q_ref[...].astype(jnp.bfloat16)
        y2 = _edge_preact(g_ref[...], q, wgs_scr[...], wns_scr[...])
        a2 = a * scale + shift
        y3 = y2.reshape(ba_, m, hf) + a2[:, None, :]
        f = _sigmoid(y3[:, :, :af])
        c = _softplus(y3[:, :, af:])
        w = w_ref[...]
        prod = f * c * (w * w)[:, :, None]
        s_blk = jnp.sum(prod, axis=1)
        s_scr[pl.ds((i - nsteps) * ba, ba), :] = s_blk
        st2_scr[0:1, :] += jnp.sum(s_blk, axis=0).reshape(1, af)
        st2_scr[1:2, :] += jnp.sum(s_blk * s_blk, axis=0).reshape(1, af)

    @pl.when(i >= 2 * nsteps)
    def _project_phase():
        j = i - 2 * nsteps
        n_count = float(s_scr.shape[0])
        mu = st2_scr[0:1, :] / n_count
        ex2 = st2_scr[1:2, :] / n_count
        var = ex2 - mu * mu
        inv = lax.rsqrt(var + _EPS)
        scale = g2_ref[...] * inv
        shift = b2_ref[...] - mu * scale
        s = s_scr[pl.ds(j * ba2, ba2), :]
        h = _softplus(atom_ref[...] + s * scale + shift)
        out_ref[...] = (
            jnp.dot(h, wp_ref[...], preferred_element_type=jnp.float32)
            + bp_ref[...]
        )


def _tc_pipeline(atom_fea, g0, nbr_flat, bond_w, W_full, b_full,
                 g1, b1, g2, b2, W_proj, b_proj):
    n, af = atom_fea.shape
    nm = g0.shape[0]
    m = nm // n
    nf = nbr_flat.shape[1]
    hf = 2 * af

    wc = W_full[:af]
    wg = W_full[af:2 * af].astype(jnp.bfloat16)
    wn = W_full[2 * af:].astype(jnp.bfloat16)

    a_center = pl.pallas_call(
        _center_body,
        out_shape=jax.ShapeDtypeStruct((n, hf), jnp.float32),
    )(atom_fea, wc, b_full.reshape(1, hf))

    ba = 200
    be = ba * m
    nsteps = n // ba
    ba2 = 2000
    k4steps = n // ba2
    total = 2 * nsteps + k4steps
    full = lambda shp: pl.BlockSpec(shp, lambda i: (0,) * len(shp))

    def edge_idx(i):
        return jnp.where(i >= 2 * nsteps, nsteps - 1, lax.rem(i, nsteps))

    atom_out = pl.pallas_call(
        functools.partial(_fused_body, nsteps, ba, ba2),
        grid=(total,),
        in_specs=[
            pl.BlockSpec((be, af), lambda i: (edge_idx(i), 0)),
            pl.BlockSpec((be, nf), lambda i: (edge_idx(i), 0)),
            pl.BlockSpec((ba, hf), lambda i: (edge_idx(i), 0)),
            pl.BlockSpec((ba, m), lambda i: (edge_idx(i), 0)),
            pl.BlockSpec((ba2, af),
                         lambda i: (jnp.maximum(i - 2 * nsteps, 0), 0)),
            full((af, hf)),
            full((nf, hf)),
            full((1, hf)),
            full((1, hf)),
            full((1, af)),
            full((1, af)),
            full((af, af)),
            full((1, af)),
        ],
        out_specs=pl.BlockSpec((ba2, af),
                               lambda i: (jnp.maximum(i - 2 * nsteps, 0), 0)),
        out_shape=jax.ShapeDtypeStruct((n, af), jnp.float32),
        scratch_shapes=[
            pltpu.VMEM((8, hf), jnp.float32),
            pltpu.VMEM((8, af), jnp.float32),
            pltpu.VMEM((af, hf), jnp.bfloat16),
            pltpu.VMEM((nf, hf), jnp.bfloat16),
            pltpu.VMEM((n, af), jnp.float32),
        ],
    )(g0, nbr_flat, a_center, bond_w, atom_fea, wg, wn,
      g1.reshape(1, hf), b1.reshape(1, hf), g2.reshape(1, af),
      b2.reshape(1, af), W_proj, b_proj.reshape(1, af))

    return atom_out


def kernel(atom_fea, nbr_fea, nbr_fea_idx, bond_weights_ag,
           W_full, b_full, g1, b1, g2, b2, W_proj, b_proj):
    n, m = nbr_fea_idx.shape
    nf = nbr_fea.shape[2]
    idx_flat = nbr_fea_idx.reshape(n * m).astype(jnp.int32)
    g0 = _sc_gather(atom_fea, idx_flat)
    nbr_flat = nbr_fea.reshape(n * m, nf).astype(jnp.bfloat16)
    atom_out = _tc_pipeline(atom_fea, g0, nbr_flat, bond_weights_ag,
                            W_full, b_full, g1, b1, g2, b2, W_proj, b_proj)
    return atom_out, nbr_fea


# speedup vs baseline: 1.1181x; 1.0024x over previous
import functools

import jax
import jax.numpy as jnp
from jax import lax
from jax.experimental import pallas as pl
from jax.experimental.pallas import tpu as pltpu
from jax.experimental.pallas import tpu_sc as plsc

_EPS = 1e-5


def _softplus(x):
    return jnp.maximum(x, 0.0) + jnp.log1p(jnp.exp(-jnp.abs(x)))


def _sigmoid(x):
    return 0.5 * jnp.tanh(0.5 * x) + 0.5


def _sc_gather(table, idx_flat, chunk=400):
    n_rows, d = table.shape
    b = idx_flat.shape[0]
    info = plsc.get_sparse_core_info()
    nw = info.num_cores * info.num_subcores
    per_w = b // nw
    while per_w % chunk != 0:
        chunk //= 2
    assert per_w * nw == b and per_w % chunk == 0 and chunk % 8 == 0
    n_chunks = per_w // chunk
    assert n_chunks >= 3
    mesh = plsc.VectorSubcoreMesh(core_axis_name="c", subcore_axis_name="s")

    @functools.partial(
        pl.kernel,
        mesh=mesh,
        out_type=jax.ShapeDtypeStruct((b, d), table.dtype),
        compiler_params=pltpu.CompilerParams(use_tc_tiling_on_sc=True),
        scratch_types=[
            pltpu.VMEM((chunk,), jnp.int32),
            pltpu.VMEM((chunk,), jnp.int32),
            pltpu.VMEM((chunk, d), table.dtype),
            pltpu.VMEM((chunk, d), table.dtype),
            pltpu.SemaphoreType.DMA,
            pltpu.SemaphoreType.DMA,
            pltpu.SemaphoreType.DMA,
            pltpu.SemaphoreType.DMA,
        ],
    )
    def k(table_hbm, idx_hbm, out_hbm, idx_v0, idx_v1, rows_v0, rows_v1,
          sem_g0, sem_g1, sem_o0, sem_o1):
        wid = lax.axis_index("s") * info.num_cores + lax.axis_index("c")
        base = pl.multiple_of(wid * per_w, 8)
        idx_v = (idx_v0, idx_v1)
        rows_v = (rows_v0, rows_v1)
        sem_g = (sem_g0, sem_g1)
        sem_o = (sem_o0, sem_o1)

        def off(ci):
            return pl.multiple_of(base + ci * chunk, 8)

        def launch(ci, sl):
            pltpu.sync_copy(idx_hbm.at[pl.ds(off(ci), chunk)], idx_v[sl])
            pltpu.async_copy(table_hbm.at[idx_v[sl]], rows_v[sl], sem_g[sl])

        launch(0, 0)
        launch(1, 1)

        def step(ci, sl, refill):
            pltpu.make_async_copy(table_hbm.at[idx_v[sl]], rows_v[sl],
                                  sem_g[sl]).wait()
            pltpu.async_copy(rows_v[sl], out_hbm.at[pl.ds(off(ci), chunk)],
                             sem_o[sl])

            if refill:
                @pl.when(ci + 2 < n_chunks)
                def _refill():
                    pltpu.make_async_copy(
                        rows_v[sl], out_hbm.at[pl.ds(off(ci), chunk)],
                        sem_o[sl]).wait()
                    launch(ci + 2, sl)

        def body(j, _):
            for sl in range(2):
                step(2 * j + sl, sl, True)
            return ()

        lax.fori_loop(0, n_chunks // 2, body, (), unroll=False)
        if n_chunks % 2:
            step(n_chunks - 1, 0, False)
        for sl in range(2):
            pltpu.make_async_copy(rows_v[sl],
                                  out_hbm.at[pl.ds(base, chunk)],
                                  sem_o[sl]).wait()

    return k(table, idx_flat)


def _center_body(atom_ref, wc_ref, bf_ref, out_ref):
    out_ref[...] = (
        jnp.dot(atom_ref[...], wc_ref[...], preferred_element_type=jnp.float32)
        + bf_ref[...]
    ).astype(jnp.bfloat16)


def _edge_preact(g, q, wg, wn):
    return (
        jnp.dot(g.astype(jnp.bfloat16), wg, preferred_element_type=jnp.float32)
        + jnp.dot(q, wn, preferred_element_type=jnp.float32)
    )


def _fused_body(nsteps, ba, ba2, g_ref, q_ref, a_ref, w_ref, atom_ref,
                wg_ref, wn_ref, g1_ref, b1_ref, g2_ref, b2_ref, wp_ref,
                bp_ref, out_ref,
                stats_scr, st2_scr, wgs_scr, wns_scr, s_scr):
    i = pl.program_id(0)

    @pl.when(i == 0)
    def _init():
        stats_scr[...] = jnp.zeros_like(stats_scr)
        st2_scr[...] = jnp.zeros_like(st2_scr)

    @pl.when(i < nsteps)
    def _stats_phase():
        a = a_ref[...].astype(jnp.float32)
        q = q_ref[...].astype(jnp.bfloat16)
        x2 = _edge_preact(g_ref[...], q, wg_ref[...], wn_ref[...])
        ba_, hf = a.shape
        m = x2.shape[0] // ba_
        t = jnp.sum(x2.reshape(ba_, m, hf), axis=1)
        s = jnp.sum(t, axis=0) + m * jnp.sum(a, axis=0)
        sq = (
            jnp.sum(x2 * x2, axis=0)
            + 2.0 * jnp.sum(a * t, axis=0)
            + m * jnp.sum(a * a, axis=0)
        )
        stats_scr[0:1, :] += s.reshape(1, hf)
        stats_scr[1:2, :] += sq.reshape(1, hf)

    @pl.when((i >= nsteps) & (i < 2 * nsteps))
    def _apply_phase():
        a = a_ref[...].astype(jnp.float32)
        ba_, hf = a.shape
        af = hf // 2
        m = g_ref.shape[0] // ba_
        count = float(nsteps * ba * m)
        mu = stats_scr[0:1, :] / count
        ex2 = stats_scr[1:2, :] / count
        var = ex2 - mu * mu
        inv = lax.rsqrt(var + _EPS)
        scale = g1_ref[...] * inv
        shift = b1_ref[...] - mu * scale

        @pl.when(i == nsteps)
        def _scale_weights():
            wgs_scr[...] = (wg_ref[...] * scale.astype(jnp.bfloat16))
            wns_scr[...] = (wn_ref[...] * scale.astype(jnp.bfloat16))

        q = q_ref[...].astype(jnp.bfloat16)
        y2 = _edge_preact(g_ref[...], q, wgs_scr[...], wns_scr[...])
        a2 = a * scale + shift
        y3 = y2.reshape(ba_, m, hf) + a2[:, None, :]
        f = _sigmoid(y3[:, :, :af])
        c = _softplus(y3[:, :, af:])
        w = w_ref[...]
        prod = f * c * (w * w)[:, :, None]
        s_blk = jnp.sum(prod, axis=1)
        s_scr[pl.ds((i - nsteps) * ba, ba), :] = s_blk
        st2_scr[0:1, :] += jnp.sum(s_blk, axis=0).reshape(1, af)
        st2_scr[1:2, :] += jnp.sum(s_blk * s_blk, axis=0).reshape(1, af)

    @pl.when(i >= 2 * nsteps)
    def _project_phase():
        j = i - 2 * nsteps
        n_count = float(s_scr.shape[0])
        mu = st2_scr[0:1, :] / n_count
        ex2 = st2_scr[1:2, :] / n_count
        var = ex2 - mu * mu
        inv = lax.rsqrt(var + _EPS)
        scale = g2_ref[...] * inv
        shift = b2_ref[...] - mu * scale
        s = s_scr[pl.ds(j * ba2, ba2), :]
        h = _softplus(atom_ref[...] + s * scale + shift)
        out_ref[...] = (
            jnp.dot(h, wp_ref[...], preferred_element_type=jnp.float32)
            + bp_ref[...]
        )


def _tc_pipeline(atom_fea, g0, nbr_flat, bond_w, W_full, b_full,
                 g1, b1, g2, b2, W_proj, b_proj):
    n, af = atom_fea.shape
    nm = g0.shape[0]
    m = nm // n
    nf = nbr_flat.shape[1]
    hf = 2 * af

    wc = W_full[:af]
    wg = W_full[af:2 * af].astype(jnp.bfloat16)
    wn = W_full[2 * af:].astype(jnp.bfloat16)

    a_center = pl.pallas_call(
        _center_body,
        out_shape=jax.ShapeDtypeStruct((n, hf), jnp.bfloat16),
    )(atom_fea, wc, b_full.reshape(1, hf))

    ba = 200
    be = ba * m
    nsteps = n // ba
    ba2 = 2000
    k4steps = n // ba2
    total = 2 * nsteps + k4steps
    full = lambda shp: pl.BlockSpec(shp, lambda i: (0,) * len(shp))

    def edge_idx(i):
        return jnp.where(i >= 2 * nsteps, nsteps - 1, lax.rem(i, nsteps))

    atom_out = pl.pallas_call(
        functools.partial(_fused_body, nsteps, ba, ba2),
        grid=(total,),
        in_specs=[
            pl.BlockSpec((be, af), lambda i: (edge_idx(i), 0)),
            pl.BlockSpec((be, nf), lambda i: (edge_idx(i), 0)),
            pl.BlockSpec((ba, hf), lambda i: (edge_idx(i), 0)),
            pl.BlockSpec((ba, m), lambda i: (edge_idx(i), 0)),
            pl.BlockSpec((ba2, af),
                         lambda i: (jnp.maximum(i - 2 * nsteps, 0), 0)),
            full((af, hf)),
            full((nf, hf)),
            full((1, hf)),
            full((1, hf)),
            full((1, af)),
            full((1, af)),
            full((af, af)),
            full((1, af)),
        ],
        out_specs=pl.BlockSpec((ba2, af),
                               lambda i: (jnp.maximum(i - 2 * nsteps, 0), 0)),
        out_shape=jax.ShapeDtypeStruct((n, af), jnp.float32),
        scratch_shapes=[
            pltpu.VMEM((8, hf), jnp.float32),
            pltpu.VMEM((8, af), jnp.float32),
            pltpu.VMEM((af, hf), jnp.bfloat16),
            pltpu.VMEM((nf, hf), jnp.bfloat16),
            pltpu.VMEM((n, af), jnp.float32),
        ],
    )(g0, nbr_flat, a_center, bond_w, atom_fea, wg, wn,
      g1.reshape(1, hf), b1.reshape(1, hf), g2.reshape(1, af),
      b2.reshape(1, af), W_proj, b_proj.reshape(1, af))

    return atom_out


def kernel(atom_fea, nbr_fea, nbr_fea_idx, bond_weights_ag,
           W_full, b_full, g1, b1, g2, b2, W_proj, b_proj):
    n, m = nbr_fea_idx.shape
    nf = nbr_fea.shape[2]
    idx_flat = nbr_fea_idx.reshape(n * m).astype(jnp.int32)
    g0 = _sc_gather(atom_fea, idx_flat)
    nbr_flat = nbr_fea.reshape(n * m, nf).astype(jnp.bfloat16)
    atom_out = _tc_pipeline(atom_fea, g0, nbr_flat, bond_weights_ag,
                            W_full, b_full, g1, b1, g2, b2, W_proj, b_proj)
    return atom_out, nbr_fea
